# baseline (speedup 1.0000x reference)
import jax
import jax.numpy as jnp
from jax.experimental import pallas as pl

_FILL = -10000.0
_VOCAB = 100000
_RESTRICTED = 65
_VB = 2048


def _body(hs_ref, wt_ref, out_ref):
    j = pl.program_id(0)
    out_ref[...] = jnp.full(out_ref.shape, _FILL, dtype=jnp.float32)

    @pl.when(j == 0)
    def _scatter():
        logits = jnp.dot(hs_ref[...], wt_ref[...],
                         preferred_element_type=jnp.float32)
        out_ref[:, 100:164] = logits[:, 0:64]
        out_ref[:, 999:1000] = logits[:, 64:65]


def kernel(hidden_states, W):
    B, T, H = hidden_states.shape
    hs = hidden_states.reshape(T, H).astype(jnp.float32)
    wt = jnp.zeros((H, 128), dtype=jnp.float32).at[:, :_RESTRICTED].set(
        W.astype(jnp.float32).T)

    n_blocks = pl.cdiv(_VOCAB, _VB)
    out = pl.pallas_call(
        _body,
        grid=(n_blocks,),
        in_specs=[
            pl.BlockSpec((T, H), lambda j: (0, 0)),
            pl.BlockSpec((H, 128), lambda j: (0, 0)),
        ],
        out_specs=pl.BlockSpec((T, _VB), lambda j: (0, j)),
        out_shape=jax.ShapeDtypeStruct((T, _VOCAB), jnp.float32),
    )(hs, wt)
    return out.reshape(B, T, _VOCAB)
